# Initial kernel scaffold; baseline (speedup 1.0000x reference)
import jax
import jax.numpy as jnp
from jax.experimental import pallas as pl


def kernel(x, edge_index, W0_self, W0_neigh, b0_self, b0_neigh, W1_self, W1_neigh, b1_self, b1_neigh, Wc, bc):
    raise NotImplementedError("write your pallas kernel here")


# speedup vs baseline: 2.5772x; 2.5772x over previous
import functools

import jax
import jax.numpy as jnp
from jax import lax
from jax.experimental import pallas as pl
from jax.experimental.pallas import tpu as pltpu
from jax.experimental.pallas import tpu_sc as plsc

_N = 10000
_E = 320000
_D = 128
_H = 128
_C = 41

_NC = 2
_NS = 16
_B = 64
_NP = 10240
_EP = 327680
_ROWS = _EP // _B
_RPW = _ROWS // (_NC * _NS)
_NPT = _NP // _NS
_ZCH = 64
_NP8 = _NP // 8


def _sc_aggregate(with_deg, y, src_r, dst_r):
    mesh = plsc.VectorSubcoreMesh(core_axis_name="c", subcore_axis_name="s")
    f32 = jnp.float32
    i32 = jnp.int32
    out_type = [jax.ShapeDtypeStruct((_NC * _NP, _D), f32)]
    scratch = [
        pltpu.VMEM_SHARED((_NP, _D), f32),
        pltpu.VMEM((_B,), i32),
        pltpu.VMEM((_B,), i32),
        pltpu.VMEM((_B,), i32),
        pltpu.VMEM((_B,), i32),
        pltpu.VMEM((_B, _D), f32),
        pltpu.SemaphoreType.DMA,
    ]
    if with_deg:
        out_type.append(jax.ShapeDtypeStruct((_NC * _NP8, _D), f32))
        scratch += [
            pltpu.VMEM_SHARED((_NP8, _D), f32),
            pltpu.VMEM_SHARED((_NS * 8, _D), f32),
            pltpu.VMEM((16, _D), f32),
            pltpu.VMEM((_B,), i32),
            pltpu.VMEM((_B,), i32),
            pltpu.VMEM((16,), i32),
            pltpu.VMEM((16,), i32),
        ]

    @functools.partial(pl.kernel, out_type=tuple(out_type), mesh=mesh,
                       scratch_types=tuple(scratch))
    def agg(*refs):
        if with_deg:
            (y_hbm, src_hbm, dst_hbm, out_sum, out_deg,
             acc, src1_v, dst1_v, ridx_v, oidx_v, rows_v, sem,
             dacc8, pat_sp, pat_tile_v, midx_v, d8idx_v, zidx_v,
             oidx16_v) = refs
        else:
            (y_hbm, src_hbm, dst_hbm, out_sum,
             acc, src1_v, dst1_v, ridx_v, oidx_v, rows_v, sem) = refs

        c = lax.axis_index("c")
        s = lax.axis_index("s")
        wid = c * _NS + s
        lane = jax.lax.iota(i32, 16)

        def fill(r, _):
            for cc in range(_D // 16):
                rows_v[r, pl.ds(cc * 16, 16)] = jnp.zeros((16,), f32)
            return 0
        lax.fori_loop(0, _B, fill, 0)
        if with_deg:
            for r in range(16):
                for q in range(8):
                    v = (jnp.ones((16,), f32) if q == r % 8
                         else jnp.zeros((16,), f32))
                    pat_tile_v[r, pl.ds(q * 16, 16)] = v
            zidx_v[...] = s * 8 + jnp.bitwise_and(lane, 7)
            pltpu.sync_copy(pat_tile_v, pat_sp.at[zidx_v])

        def zbody(k, _):
            base = s * _NPT + k * _B
            for q in range(_B // 16):
                ridx_v[pl.ds(q * 16, 16)] = base + q * 16 + lane
            pltpu.sync_copy(rows_v, acc.at[ridx_v])
            return 0
        lax.fori_loop(0, _NPT // _B, zbody, 0)
        if with_deg:
            def z8body(k, _):
                zidx_v[...] = s * (_NP8 // _NS) + k * 16 + lane
                pltpu.sync_copy(rows_v.at[pl.ds(0, 16)], dacc8.at[zidx_v])
                return 0
            lax.fori_loop(0, _NP8 // _NS // 16, z8body, 0)
        plsc.subcore_barrier()

        def body(j, _):
            base = (wid * _RPW + j) * _B
            pltpu.sync_copy(src_hbm.at[pl.ds(base, _B)], src1_v)
            pltpu.sync_copy(dst_hbm.at[pl.ds(base, _B)], dst1_v)
            pltpu.async_copy(y_hbm.at[src1_v], rows_v, sem).wait()
            pltpu.sync_copy(rows_v, acc.at[dst1_v], add=True)
            if with_deg:
                for q in range(_B // 16):
                    d = dst1_v[pl.ds(q * 16, 16)]
                    d8idx_v[pl.ds(q * 16, 16)] = (
                        lax.shift_right_logical(d, 3))
                    midx_v[pl.ds(q * 16, 16)] = (
                        jnp.bitwise_and(d, 7) + s * 8)
                pltpu.async_copy(pat_sp.at[midx_v], rows_v, sem).wait()
                pltpu.sync_copy(rows_v, dacc8.at[d8idx_v], add=True)
            return 0
        lax.fori_loop(0, _RPW, body, 0)

        plsc.subcore_barrier()

        def obody(k, _):
            base = s * _NPT + k * _B
            for q in range(_B // 16):
                idx = base + q * 16 + lane
                ridx_v[pl.ds(q * 16, 16)] = idx
                oidx_v[pl.ds(q * 16, 16)] = c * _NP + idx
            pltpu.async_copy(acc.at[ridx_v], rows_v, sem).wait()
            pltpu.sync_copy(rows_v, out_sum.at[oidx_v])
            return 0
        lax.fori_loop(0, _NPT // _B, obody, 0)
        if with_deg:
            def d8body(k, _):
                idx = s * (_NP8 // _NS) + k * 16 + lane
                zidx_v[...] = idx
                oidx16_v[...] = c * _NP8 + idx
                pltpu.async_copy(dacc8.at[zidx_v],
                                 rows_v.at[pl.ds(0, 16)], sem).wait()
                pltpu.sync_copy(rows_v.at[pl.ds(0, 16)],
                                out_deg.at[oidx16_v])
                return 0
            lax.fori_loop(0, _NP8 // _NS // 16, d8body, 0)

    return agg(y, src_r, dst_r)


_BLK = 1000


def _tc_a_body(x_ref, w0s_ref, b0s_ref, w0n_ref, hs_ref, y0_ref):
    x = x_ref[...]
    hs_ref[...] = jnp.maximum(
        jnp.dot(x, w0s_ref[...], preferred_element_type=jnp.float32)
        + b0s_ref[...], 0.0)
    y0_ref[...] = jnp.dot(x, w0n_ref[...], preferred_element_type=jnp.float32)


def _tc_b_body(hs_ref, p0_ref, p1_ref, d0_ref, d1_ref, b0n_ref,
               w1s_ref, b1s_ref, w1n_ref, z1s_ref, y1_ref):
    deg = jnp.maximum(d0_ref[:, 0:1] + d1_ref[:, 0:1], 1.0)
    hn = jnp.maximum((p0_ref[...] + p1_ref[...]) / deg + b0n_ref[...], 0.0)
    h0 = jnp.concatenate([hs_ref[...], hn], axis=1)
    z1s_ref[...] = jnp.maximum(
        jnp.dot(h0, w1s_ref[...], preferred_element_type=jnp.float32)
        + b1s_ref[...], 0.0)
    y1_ref[...] = jnp.dot(h0, w1n_ref[...], preferred_element_type=jnp.float32)


def _tc_c_body(z1s_ref, q0_ref, q1_ref, d0_ref, d1_ref, b1n_ref,
               wc_ref, bc_ref, out_ref):
    deg = jnp.maximum(d0_ref[:, 0:1] + d1_ref[:, 0:1], 1.0)
    hn = jnp.maximum((q0_ref[...] + q1_ref[...]) / deg + b1n_ref[...], 0.0)
    h1 = jnp.concatenate([z1s_ref[...], hn], axis=1)
    norm = jnp.maximum(
        jnp.sqrt(jnp.sum(h1 * h1, axis=1, keepdims=True)), 1e-12)
    out_ref[...] = (jnp.dot(h1 / norm, wc_ref[...],
                            preferred_element_type=jnp.float32)
                    + bc_ref[...])


def _row_spec(cols):
    return pl.BlockSpec((_BLK, cols), lambda i: (i, 0))


def _full_spec(rows, cols):
    return pl.BlockSpec((rows, cols), lambda i: (0, 0))


def kernel(x, edge_index, W0_self, W0_neigh, b0_self, b0_neigh,
           W1_self, W1_neigh, b1_self, b1_neigh, Wc, bc):
    f32 = jnp.float32
    grid = (_N // _BLK,)
    pad = _EP - _E
    src_p = jnp.concatenate([edge_index[0], jnp.zeros((pad,), jnp.int32)])
    dst_p = jnp.concatenate([edge_index[1], jnp.full((pad,), _N, jnp.int32)])
    src_r = src_p
    dst_r = dst_p
    b0s = b0_self.reshape(1, _H)
    b0n = b0_neigh.reshape(1, _H)
    b1s = b1_self.reshape(1, _H)
    b1n = b1_neigh.reshape(1, _H)
    bcr = bc.reshape(1, _C)

    hs0, y0 = pl.pallas_call(
        _tc_a_body,
        grid=grid,
        in_specs=[_row_spec(_D), _full_spec(_D, _H), _full_spec(1, _H),
                  _full_spec(_D, _H)],
        out_specs=[_row_spec(_H), _row_spec(_H)],
        out_shape=[jax.ShapeDtypeStruct((_N, _H), f32),
                   jax.ShapeDtypeStruct((_N, _H), f32)],
    )(x, W0_self, b0s, W0_neigh)

    p, dp = _sc_aggregate(True, y0, src_r, dst_r)
    p = p.reshape(_NC, _NP, _D)
    dp = dp.reshape(_NC, _NP, 16)

    z1s, y1 = pl.pallas_call(
        _tc_b_body,
        grid=grid,
        in_specs=[_row_spec(_H), _row_spec(_H), _row_spec(_H),
                  _row_spec(16), _row_spec(16), _full_spec(1, _H),
                  _full_spec(2 * _H, _H), _full_spec(1, _H),
                  _full_spec(2 * _H, _H)],
        out_specs=[_row_spec(_H), _row_spec(_H)],
        out_shape=[jax.ShapeDtypeStruct((_N, _H), f32),
                   jax.ShapeDtypeStruct((_N, _H), f32)],
    )(hs0, p[0], p[1], dp[0], dp[1], b0n, W1_self, b1s, W1_neigh)

    (q,) = _sc_aggregate(False, y1, src_r, dst_r)
    q = q.reshape(_NC, _NP, _D)

    logits = pl.pallas_call(
        _tc_c_body,
        grid=grid,
        in_specs=[_row_spec(_H), _row_spec(_H), _row_spec(_H),
                  _row_spec(16), _row_spec(16), _full_spec(1, _H),
                  _full_spec(2 * _H, _C), _full_spec(1, _C)],
        out_specs=[_row_spec(_C)],
        out_shape=[jax.ShapeDtypeStruct((_N, _C), f32)],
    )(z1s, q[0], q[1], dp[0], dp[1], b1n, Wc, bcr)[0]

    return logits


# speedup vs baseline: 3.0686x; 1.1907x over previous
import functools

import jax
import jax.numpy as jnp
from jax import lax
from jax.experimental import pallas as pl
from jax.experimental.pallas import tpu as pltpu
from jax.experimental.pallas import tpu_sc as plsc

_N = 10000
_E = 320000
_D = 128
_H = 128
_C = 41

_NC = 2
_NS = 16
_B = 64
_NP = 10240
_EP = 327680
_ROWS = _EP // _B
_RPW = _ROWS // (_NC * _NS)
_NPT = _NP // _NS
_ZCH = 64
_NP8 = _NP // 8


def _sc_aggregate(with_deg, y, src_r, dst_r):
    mesh = plsc.VectorSubcoreMesh(core_axis_name="c", subcore_axis_name="s")
    f32 = jnp.float32
    i32 = jnp.int32
    out_type = [jax.ShapeDtypeStruct((_NC * _NP, _D), f32)]
    scratch = [
        pltpu.VMEM_SHARED((_NP, _D), f32),
        pltpu.VMEM((_B,), i32),
        pltpu.VMEM((_B,), i32),
        pltpu.VMEM((_B,), i32),
        pltpu.VMEM((_B,), i32),
        pltpu.VMEM((_B,), i32),
        pltpu.VMEM((_B,), i32),
        pltpu.VMEM((_B, _D), f32),
        pltpu.VMEM((_B, _D), f32),
        pltpu.SemaphoreType.DMA,
        pltpu.SemaphoreType.DMA,
    ]
    if with_deg:
        out_type.append(jax.ShapeDtypeStruct((_NC * _NP8, _D), f32))
        scratch += [
            pltpu.VMEM_SHARED((_NP8, _D), f32),
            pltpu.VMEM_SHARED((_NS * 8, _D), f32),
            pltpu.VMEM((16, _D), f32),
            pltpu.VMEM((_B,), i32),
            pltpu.VMEM((_B,), i32),
            pltpu.VMEM((16,), i32),
            pltpu.VMEM((16,), i32),
        ]

    @functools.partial(pl.kernel, out_type=tuple(out_type), mesh=mesh,
                       scratch_types=tuple(scratch))
    def agg(*refs):
        if with_deg:
            (y_hbm, src_hbm, dst_hbm, out_sum, out_deg,
             acc, src1_v, dst1_v, src2_v, dst2_v, ridx_v, oidx_v,
             rows_v, rows2_v, sem, sem2,
             dacc8, pat_sp, pat_tile_v, midx_v, d8idx_v, zidx_v,
             oidx16_v) = refs
        else:
            (y_hbm, src_hbm, dst_hbm, out_sum,
             acc, src1_v, dst1_v, src2_v, dst2_v, ridx_v, oidx_v,
             rows_v, rows2_v, sem, sem2) = refs

        c = lax.axis_index("c")
        s = lax.axis_index("s")
        wid = c * _NS + s
        lane = jax.lax.iota(i32, 16)

        def fill(r, _):
            for cc in range(_D // 16):
                rows_v[r, pl.ds(cc * 16, 16)] = jnp.zeros((16,), f32)
            return 0
        lax.fori_loop(0, _B, fill, 0)
        if with_deg:
            for r in range(16):
                for q in range(8):
                    v = (jnp.ones((16,), f32) if q == r % 8
                         else jnp.zeros((16,), f32))
                    pat_tile_v[r, pl.ds(q * 16, 16)] = v
            zidx_v[...] = s * 8 + jnp.bitwise_and(lane, 7)
            pltpu.sync_copy(pat_tile_v, pat_sp.at[zidx_v])

        def zbody(k, _):
            base = s * _NPT + k * _B
            for q in range(_B // 16):
                ridx_v[pl.ds(q * 16, 16)] = base + q * 16 + lane
            pltpu.sync_copy(rows_v, acc.at[ridx_v])
            return 0
        lax.fori_loop(0, _NPT // _B, zbody, 0)
        if with_deg:
            def z8body(k, _):
                zidx_v[...] = s * (_NP8 // _NS) + k * 16 + lane
                pltpu.sync_copy(rows_v.at[pl.ds(0, 16)], dacc8.at[zidx_v])
                return 0
            lax.fori_loop(0, _NP8 // _NS // 16, z8body, 0)
        plsc.subcore_barrier()

        def body(j, _):
            base = (wid * _RPW + 2 * j) * _B
            pltpu.sync_copy(src_hbm.at[pl.ds(base, _B)], src1_v)
            pltpu.sync_copy(dst_hbm.at[pl.ds(base, _B)], dst1_v)
            cpa = pltpu.async_copy(y_hbm.at[src1_v], rows_v, sem)
            pltpu.sync_copy(src_hbm.at[pl.ds(base + _B, _B)], src2_v)
            pltpu.sync_copy(dst_hbm.at[pl.ds(base + _B, _B)], dst2_v)
            cpb = pltpu.async_copy(y_hbm.at[src2_v], rows2_v, sem2)
            cpa.wait()
            pltpu.sync_copy(rows_v, acc.at[dst1_v], add=True)
            if with_deg:
                for q in range(_B // 16):
                    d = dst1_v[pl.ds(q * 16, 16)]
                    d8idx_v[pl.ds(q * 16, 16)] = (
                        lax.shift_right_logical(d, 3))
                    midx_v[pl.ds(q * 16, 16)] = (
                        jnp.bitwise_and(d, 7) + s * 8)
                pltpu.async_copy(pat_sp.at[midx_v], rows_v, sem).wait()
                pltpu.sync_copy(rows_v, dacc8.at[d8idx_v], add=True)
            cpb.wait()
            pltpu.sync_copy(rows2_v, acc.at[dst2_v], add=True)
            if with_deg:
                for q in range(_B // 16):
                    d = dst2_v[pl.ds(q * 16, 16)]
                    d8idx_v[pl.ds(q * 16, 16)] = (
                        lax.shift_right_logical(d, 3))
                    midx_v[pl.ds(q * 16, 16)] = (
                        jnp.bitwise_and(d, 7) + s * 8)
                pltpu.async_copy(pat_sp.at[midx_v], rows2_v, sem2).wait()
                pltpu.sync_copy(rows2_v, dacc8.at[d8idx_v], add=True)
            return 0
        lax.fori_loop(0, _RPW // 2, body, 0)

        plsc.subcore_barrier()

        def obody(k, _):
            base = s * _NPT + k * _B
            for q in range(_B // 16):
                idx = base + q * 16 + lane
                ridx_v[pl.ds(q * 16, 16)] = idx
                oidx_v[pl.ds(q * 16, 16)] = c * _NP + idx
            pltpu.async_copy(acc.at[ridx_v], rows_v, sem).wait()
            pltpu.sync_copy(rows_v, out_sum.at[oidx_v])
            return 0
        lax.fori_loop(0, _NPT // _B, obody, 0)
        if with_deg:
            def d8body(k, _):
                idx = s * (_NP8 // _NS) + k * 16 + lane
                zidx_v[...] = idx
                oidx16_v[...] = c * _NP8 + idx
                pltpu.async_copy(dacc8.at[zidx_v],
                                 rows_v.at[pl.ds(0, 16)], sem).wait()
                pltpu.sync_copy(rows_v.at[pl.ds(0, 16)],
                                out_deg.at[oidx16_v])
                return 0
            lax.fori_loop(0, _NP8 // _NS // 16, d8body, 0)

    return agg(y, src_r, dst_r)


_BLK = 1000


def _tc_a_body(x_ref, w0s_ref, b0s_ref, w0n_ref, hs_ref, y0_ref):
    x = x_ref[...]
    hs_ref[...] = jnp.maximum(
        jnp.dot(x, w0s_ref[...], preferred_element_type=jnp.float32)
        + b0s_ref[...], 0.0)
    y0_ref[...] = jnp.dot(x, w0n_ref[...], preferred_element_type=jnp.float32)


def _tc_b_body(hs_ref, p0_ref, p1_ref, d0_ref, d1_ref, b0n_ref,
               w1s_ref, b1s_ref, w1n_ref, z1s_ref, y1_ref):
    deg = jnp.maximum(d0_ref[:, 0:1] + d1_ref[:, 0:1], 1.0)
    hn = jnp.maximum((p0_ref[...] + p1_ref[...]) / deg + b0n_ref[...], 0.0)
    h0 = jnp.concatenate([hs_ref[...], hn], axis=1)
    z1s_ref[...] = jnp.maximum(
        jnp.dot(h0, w1s_ref[...], preferred_element_type=jnp.float32)
        + b1s_ref[...], 0.0)
    y1_ref[...] = jnp.dot(h0, w1n_ref[...], preferred_element_type=jnp.float32)


def _tc_c_body(z1s_ref, q0_ref, q1_ref, d0_ref, d1_ref, b1n_ref,
               wc_ref, bc_ref, out_ref):
    deg = jnp.maximum(d0_ref[:, 0:1] + d1_ref[:, 0:1], 1.0)
    hn = jnp.maximum((q0_ref[...] + q1_ref[...]) / deg + b1n_ref[...], 0.0)
    h1 = jnp.concatenate([z1s_ref[...], hn], axis=1)
    norm = jnp.maximum(
        jnp.sqrt(jnp.sum(h1 * h1, axis=1, keepdims=True)), 1e-12)
    out_ref[...] = (jnp.dot(h1 / norm, wc_ref[...],
                            preferred_element_type=jnp.float32)
                    + bc_ref[...])


def _row_spec(cols):
    return pl.BlockSpec((_BLK, cols), lambda i: (i, 0))


def _full_spec(rows, cols):
    return pl.BlockSpec((rows, cols), lambda i: (0, 0))


def kernel(x, edge_index, W0_self, W0_neigh, b0_self, b0_neigh,
           W1_self, W1_neigh, b1_self, b1_neigh, Wc, bc):
    f32 = jnp.float32
    grid = (_N // _BLK,)
    pad = _EP - _E
    src_p = jnp.concatenate([edge_index[0], jnp.zeros((pad,), jnp.int32)])
    dst_p = jnp.concatenate([edge_index[1], jnp.full((pad,), _N, jnp.int32)])
    src_r = src_p
    dst_r = dst_p
    b0s = b0_self.reshape(1, _H)
    b0n = b0_neigh.reshape(1, _H)
    b1s = b1_self.reshape(1, _H)
    b1n = b1_neigh.reshape(1, _H)
    bcr = bc.reshape(1, _C)

    hs0, y0 = pl.pallas_call(
        _tc_a_body,
        grid=grid,
        in_specs=[_row_spec(_D), _full_spec(_D, _H), _full_spec(1, _H),
                  _full_spec(_D, _H)],
        out_specs=[_row_spec(_H), _row_spec(_H)],
        out_shape=[jax.ShapeDtypeStruct((_N, _H), f32),
                   jax.ShapeDtypeStruct((_N, _H), f32)],
    )(x, W0_self, b0s, W0_neigh)

    p, dp = _sc_aggregate(True, y0, src_r, dst_r)
    p = p.reshape(_NC, _NP, _D)
    dp = dp.reshape(_NC, _NP, 16)

    z1s, y1 = pl.pallas_call(
        _tc_b_body,
        grid=grid,
        in_specs=[_row_spec(_H), _row_spec(_H), _row_spec(_H),
                  _row_spec(16), _row_spec(16), _full_spec(1, _H),
                  _full_spec(2 * _H, _H), _full_spec(1, _H),
                  _full_spec(2 * _H, _H)],
        out_specs=[_row_spec(_H), _row_spec(_H)],
        out_shape=[jax.ShapeDtypeStruct((_N, _H), f32),
                   jax.ShapeDtypeStruct((_N, _H), f32)],
    )(hs0, p[0], p[1], dp[0], dp[1], b0n, W1_self, b1s, W1_neigh)

    (q,) = _sc_aggregate(False, y1, src_r, dst_r)
    q = q.reshape(_NC, _NP, _D)

    logits = pl.pallas_call(
        _tc_c_body,
        grid=grid,
        in_specs=[_row_spec(_H), _row_spec(_H), _row_spec(_H),
                  _row_spec(16), _row_spec(16), _full_spec(1, _H),
                  _full_spec(2 * _H, _C), _full_spec(1, _C)],
        out_specs=[_row_spec(_C)],
        out_shape=[jax.ShapeDtypeStruct((_N, _C), f32)],
    )(z1s, q[0], q[1], dp[0], dp[1], b1n, Wc, bcr)[0]

    return logits
